# baseline (speedup 1.0000x reference)
import functools

import jax
import jax.numpy as jnp
from jax import lax
from jax.experimental import pallas as pl
from jax.experimental.pallas import tpu as pltpu
from jax.experimental.pallas import tpu_sc as plsc

_NUM_FACTORS = 50000
_NUM_VARS = 100000
_S = 16
_E = 1600000
_LN_ZERO = -99.0
_ALPHA = 0.5

_NW = 32
_CHUNK = 128
_CH = _E // _CHUNK
_BASE = _CH // _NW
_EXTRA = _CH % _NW

def _mesh():
    return plsc.VectorSubcoreMesh(core_axis_name="c", subcore_axis_name="s")


_SC_PARAMS = pltpu.CompilerParams(
    use_tc_tiling_on_sc=False, needs_layout_passes=False)


def _worker_range(c, s):
    wid = s * 2 + c
    nch = _BASE + (wid < _EXTRA).astype(jnp.int32)
    start = wid * _BASE + jnp.minimum(wid, _EXTRA)
    return start, nch


_K = 16


def _iota16():
    return lax.iota(jnp.int32, 16)


def _sc_gather_t(table, idx2):
    @functools.partial(
        pl.kernel,
        mesh=_mesh(),
        out_type=jax.ShapeDtypeStruct((_S, _E), jnp.float32),
        scratch_types=[
            pltpu.VMEM((_K, _CHUNK), jnp.int32),
            pltpu.VMEM((_K, _CHUNK, _S), jnp.float32),
            pltpu.VMEM((_S, _K * _CHUNK), jnp.float32),
            pltpu.SemaphoreType.DMA,
            pltpu.SemaphoreType.DMA,
        ],
        compiler_params=_SC_PARAMS,
    )
    def gk(table_hbm, idx_hbm, out_hbm, idx_v, rows_v, tbuf, gsem, wsem):
        start, nch = _worker_range(lax.axis_index("c"), lax.axis_index("s"))
        nsup = nch // _K

        def transpose_chunk(j, base):
            def tg(i, carry):
                g = i
                rowi = _iota16() + g * 16
                for s in range(_S):
                    coli = _iota16() * 0 + s
                    v = plsc.load_gather(rows_v.at[j], [rowi, coli])
                    tbuf[s, pl.ds(base + g * 16, 16)] = v
                return carry
            lax.fori_loop(0, 8, tg, jnp.int32(0))

        def sup(t, carry):
            cr = start + t * _K
            pltpu.sync_copy(idx_hbm.at[pl.ds(cr, _K)], idx_v)
            descs = [
                pltpu.async_copy(
                    table_hbm.at[idx_v.at[j]], rows_v.at[j], gsem)
                for j in range(_K)
            ]

            @pl.when(t > 0)
            def _wait_prev_write():
                pltpu.make_async_copy(
                    tbuf, out_hbm.at[:, pl.ds(0, _K * _CHUNK)], wsem).wait()

            for j, d in enumerate(descs):
                d.wait()
                transpose_chunk(j, j * _CHUNK)
            pltpu.async_copy(
                tbuf, out_hbm.at[:, pl.ds(cr * _CHUNK, _K * _CHUNK)], wsem)
            return carry

        lax.fori_loop(0, nsup, sup, jnp.int32(0))

        @pl.when(nsup > 0)
        def _drain_last_write():
            pltpu.make_async_copy(
                tbuf, out_hbm.at[:, pl.ds(0, _K * _CHUNK)], wsem).wait()

        def tail(i, carry):
            cr = start + nsup * _K + i
            pltpu.sync_copy(idx_hbm.at[pl.ds(cr, 1)], idx_v.at[pl.ds(0, 1)])
            pltpu.async_copy(
                table_hbm.at[idx_v.at[0]], rows_v.at[0], gsem).wait()
            transpose_chunk(0, 0)
            pltpu.sync_copy(tbuf.at[:, pl.ds(0, _CHUNK)],
                            out_hbm.at[:, pl.ds(cr * _CHUNK, _CHUNK)])
            return carry

        lax.fori_loop(0, nch - nsup * _K, tail, jnp.int32(0))

    return gk(table, idx2)


def _sc_scatter_add_t(msgs_t, idx2, n_rows, zeros, k):
    span = n_rows // 16

    @functools.partial(
        pl.kernel,
        mesh=_mesh(),
        out_type=jax.ShapeDtypeStruct((2, n_rows, _S), jnp.float32),
        scratch_types=[
            pltpu.VMEM((k, _CHUNK), jnp.int32),
            pltpu.VMEM((k, _CHUNK, _S), jnp.float32),
            pltpu.VMEM((_S, k * _CHUNK), jnp.float32),
            pltpu.VMEM_SHARED((n_rows, _S), jnp.float32),
            pltpu.SemaphoreType.DMA,
        ],
        compiler_params=_SC_PARAMS,
    )
    def sk(msgs_hbm, idx_hbm, zeros_hbm, out_hbm,
           idx_v, rows_v, tbuf, accum, asem):
        c = lax.axis_index("c")
        s = lax.axis_index("s")
        start, nch = _worker_range(c, s)
        nsup = nch // k
        pltpu.sync_copy(zeros_hbm.at[pl.ds(s * span, span)],
                        accum.at[pl.ds(s * span, span)])
        plsc.subcore_barrier()

        def untranspose_chunk(j, base):
            def tg(i, carry):
                g = i
                rowi = _iota16() + g * 16
                for st in range(_S):
                    v = tbuf[st, pl.ds(base + g * 16, 16)]
                    coli = _iota16() * 0 + st
                    plsc.store_scatter(rows_v.at[j], [rowi, coli], v)
                return carry
            lax.fori_loop(0, 8, tg, jnp.int32(0))

        def sup(t, carry):
            cr = start + t * k
            pltpu.sync_copy(idx_hbm.at[pl.ds(cr, k)], idx_v)
            pltpu.sync_copy(
                msgs_hbm.at[:, pl.ds(cr * _CHUNK, k * _CHUNK)], tbuf)
            descs = []
            for j in range(k):
                untranspose_chunk(j, j * _CHUNK)
                descs.append(pltpu.async_copy(
                    rows_v.at[j], accum.at[idx_v.at[j]], asem, add=True))
            for d in descs:
                d.wait()
            return carry

        lax.fori_loop(0, nsup, sup, jnp.int32(0))

        def tail(i, carry):
            cr = start + nsup * k + i
            pltpu.sync_copy(idx_hbm.at[pl.ds(cr, 1)], idx_v.at[pl.ds(0, 1)])
            pltpu.sync_copy(
                msgs_hbm.at[:, pl.ds(cr * _CHUNK, _CHUNK)],
                tbuf.at[:, pl.ds(0, _CHUNK)])
            untranspose_chunk(0, 0)
            pltpu.async_copy(
                rows_v.at[0], accum.at[idx_v.at[0]], asem, add=True).wait()
            return carry

        lax.fori_loop(0, nch - nsup * k, tail, jnp.int32(0))
        plsc.subcore_barrier()
        pltpu.sync_copy(accum.at[pl.ds(s * span, span)],
                        out_hbm.at[c, pl.ds(s * span, span)])

    return sk(msgs_t, idx2, zeros)


def _dense_pass_t(at, bt, rt, w, bias):
    bc = 12800
    res_from_x = rt is None

    def body(*refs):
        refs = list(refs)
        a_ref = refs.pop(0)
        b_ref = refs.pop(0)
        r_ref = None if res_from_x else refs.pop(0)
        w_ref, bias_ref, o_ref = refs
        x = a_ref[...] - b_ref[...]
        y = jnp.dot(w_ref[...], x, preferred_element_type=jnp.float32)
        y = y + bias_ref[...]
        res = x if res_from_x else r_ref[...]
        m = jnp.maximum(_ALPHA * y + (1.0 - _ALPHA) * res, _LN_ZERO)
        mx = jnp.max(m, axis=0, keepdims=True)
        e = jnp.exp(m - mx)
        ssum = jnp.sum(e, axis=0, keepdims=True)
        o_ref[...] = m - mx - jnp.log(ssum)

    spec = pl.BlockSpec((_S, bc), lambda i: (0, i))
    in_specs = [spec, spec]
    operands = [at, bt]
    if not res_from_x:
        in_specs.append(spec)
        operands.append(rt)
    in_specs += [
        pl.BlockSpec((_S, _S), lambda i: (0, 0)),
        pl.BlockSpec((_S, 1), lambda i: (0, 0)),
    ]
    operands += [w, bias]
    return pl.pallas_call(
        body,
        grid=(_E // bc,),
        in_specs=in_specs,
        out_specs=spec,
        out_shape=jax.ShapeDtypeStruct((_S, _E), jnp.float32),
    )(*operands)


def _add_pair(p):
    n_rows = p.shape[1]
    a = p[0].reshape(n_rows * _S // 128, 128)
    b = p[1].reshape(n_rows * _S // 128, 128)
    rows = a.shape[0]
    br = 512
    grid = (rows + br - 1) // br

    def body(a_ref, b_ref, o_ref):
        o_ref[...] = a_ref[...] + b_ref[...]

    spec = pl.BlockSpec((br, 128), lambda i: (i, 0))
    out = pl.pallas_call(
        body,
        grid=(grid,),
        in_specs=[spec, spec],
        out_specs=spec,
        out_shape=jax.ShapeDtypeStruct((rows, 128), jnp.float32),
    )(a, b)
    return out.reshape(n_rows, _S)


def kernel(prv_varToFactor_messages, prv_factorToVar_messages, prv_factor_beliefs,
           W1, b1, W2, b2, W3, b3, W4, b4,
           factor_edge_idx, var_edge_idx):
    f_idx2 = factor_edge_idx.astype(jnp.int32).reshape(_CH, _CHUNK)
    v_idx2 = var_edge_idx.astype(jnp.int32).reshape(_CH, _CHUNK)

    wc1 = W2 @ W1
    bc1 = (b1 @ W2.T + b2)[:, None]
    wc2 = W4 @ W3
    bc2 = (b3 @ W4.T + b4)[:, None]

    pvtf_t = jnp.transpose(prv_varToFactor_messages)
    pftv_t = jnp.transpose(prv_factorToVar_messages)

    fb_t = _sc_gather_t(prv_factor_beliefs, f_idx2)

    ftv_t = _dense_pass_t(fb_t, pvtf_t, pftv_t, wc1, bc1)
    factorToVar_messages = jnp.transpose(ftv_t)

    vz = jnp.zeros((_NUM_VARS, _S), jnp.float32)
    vparts = _sc_scatter_add_t(ftv_t, v_idx2, _NUM_VARS, vz, 7)
    var_beliefs = _add_pair(vparts)

    vb_t = _sc_gather_t(var_beliefs, v_idx2)

    vtf_t = _dense_pass_t(vb_t, ftv_t, None, wc2, bc2)
    varToFactor_messages = jnp.transpose(vtf_t)

    fz = jnp.zeros((_NUM_FACTORS, _S), jnp.float32)
    fparts = _sc_scatter_add_t(vtf_t, f_idx2, _NUM_FACTORS, fz, 16)
    factor_beliefs = _add_pair(fparts)

    return (varToFactor_messages, factorToVar_messages, factor_beliefs, var_beliefs)


# speedup vs baseline: 3.2349x; 3.2349x over previous
import functools

import jax
import jax.numpy as jnp
from jax import lax
from jax.experimental import pallas as pl
from jax.experimental.pallas import tpu as pltpu
from jax.experimental.pallas import tpu_sc as plsc

_NUM_FACTORS = 50000
_NUM_VARS = 100000
_S = 16
_E = 1600000
_LN_ZERO = -99.0
_ALPHA = 0.5

_NW = 32
_CHUNK = 128
_CH = _E // _CHUNK
_BASE = _CH // _NW
_EXTRA = _CH % _NW


def _mesh():
    return plsc.VectorSubcoreMesh(core_axis_name="c", subcore_axis_name="s")


_SC_PARAMS = pltpu.CompilerParams(use_tc_tiling_on_sc=False)


def _worker_range(c, s):
    wid = s * 2 + c
    nch = _BASE + (wid < _EXTRA).astype(jnp.int32)
    start = wid * _BASE + jnp.minimum(wid, _EXTRA)
    return start, nch


_K = 16


def _sc_gather(table, idx2):
    @functools.partial(
        pl.kernel,
        mesh=_mesh(),
        out_type=jax.ShapeDtypeStruct((_CH, _CHUNK, _S), jnp.float32),
        scratch_types=[
            pltpu.VMEM((_K, _CHUNK), jnp.int32),
            pltpu.VMEM((_K, _CHUNK, _S), jnp.float32),
            pltpu.SemaphoreType.DMA,
            pltpu.SemaphoreType.DMA,
        ],
        compiler_params=_SC_PARAMS,
    )
    def gk(table_hbm, idx_hbm, out_hbm, idx_v, rows_v, gsem, wsem):
        start, nch = _worker_range(lax.axis_index("c"), lax.axis_index("s"))
        nsup = nch // _K

        def sup(t, carry):
            cr = start + t * _K
            pltpu.sync_copy(idx_hbm.at[pl.ds(cr, _K)], idx_v)

            @pl.when(t > 0)
            def _wait_prev_write():
                pltpu.make_async_copy(
                    rows_v, out_hbm.at[pl.ds(cr, _K)], wsem).wait()

            descs = [
                pltpu.async_copy(
                    table_hbm.at[idx_v.at[j]], rows_v.at[j], gsem)
                for j in range(_K)
            ]
            for d in descs:
                d.wait()
            pltpu.async_copy(rows_v, out_hbm.at[pl.ds(cr, _K)], wsem)
            return carry

        lax.fori_loop(0, nsup, sup, jnp.int32(0))

        @pl.when(nsup > 0)
        def _drain_last_write():
            pltpu.make_async_copy(
                rows_v, out_hbm.at[pl.ds(start, _K)], wsem).wait()

        def tail(i, carry):
            cr = start + nsup * _K + i
            pltpu.sync_copy(idx_hbm.at[pl.ds(cr, 1)], idx_v.at[pl.ds(0, 1)])
            pltpu.async_copy(
                table_hbm.at[idx_v.at[0]], rows_v.at[0], gsem).wait()
            pltpu.sync_copy(rows_v.at[pl.ds(0, 1)], out_hbm.at[pl.ds(cr, 1)])
            return carry

        lax.fori_loop(0, nch - nsup * _K, tail, jnp.int32(0))

    return gk(table, idx2)


def _sc_scatter_add(msgs3, idx2, n_rows, zeros, k):
    span = n_rows // 16

    @functools.partial(
        pl.kernel,
        mesh=_mesh(),
        out_type=jax.ShapeDtypeStruct((2, n_rows, _S), jnp.float32),
        scratch_types=[
            pltpu.VMEM((k, _CHUNK), jnp.int32),
            pltpu.VMEM((k, _CHUNK, _S), jnp.float32),
            pltpu.VMEM_SHARED((n_rows, _S), jnp.float32),
            pltpu.SemaphoreType.DMA,
        ],
        compiler_params=_SC_PARAMS,
    )
    def sk(msgs_hbm, idx_hbm, zeros_hbm, out_hbm, idx_v, rows_v, accum, asem):
        c = lax.axis_index("c")
        s = lax.axis_index("s")
        start, nch = _worker_range(c, s)
        nsup = nch // k
        pltpu.sync_copy(zeros_hbm.at[pl.ds(s * span, span)],
                        accum.at[pl.ds(s * span, span)])
        plsc.subcore_barrier()

        def sup(t, carry):
            cr = start + t * k
            pltpu.sync_copy(idx_hbm.at[pl.ds(cr, k)], idx_v)
            pltpu.sync_copy(msgs_hbm.at[pl.ds(cr, k)], rows_v)
            descs = [
                pltpu.async_copy(
                    rows_v.at[j], accum.at[idx_v.at[j]], asem, add=True)
                for j in range(k)
            ]
            for d in descs:
                d.wait()
            return carry

        lax.fori_loop(0, nsup, sup, jnp.int32(0))

        def tail(i, carry):
            cr = start + nsup * k + i
            pltpu.sync_copy(idx_hbm.at[pl.ds(cr, 1)], idx_v.at[pl.ds(0, 1)])
            pltpu.sync_copy(msgs_hbm.at[pl.ds(cr, 1)], rows_v.at[pl.ds(0, 1)])
            pltpu.async_copy(
                rows_v.at[0], accum.at[idx_v.at[0]], asem, add=True).wait()
            return carry

        lax.fori_loop(0, nch - nsup * k, tail, jnp.int32(0))
        plsc.subcore_barrier()
        pltpu.sync_copy(accum.at[pl.ds(s * span, span)],
                        out_hbm.at[c, pl.ds(s * span, span)])

    return sk(msgs3, idx2, zeros)


def _dense_pass(a2d, b2d, r2d, w, bias):
    rows = a2d.shape[0]
    br = 2000
    res_from_x = r2d is None

    def body(*refs):
        refs = list(refs)
        a_ref = refs.pop(0)
        b_ref = refs.pop(0)
        r_ref = None if res_from_x else refs.pop(0)
        w_ref, bias_ref, o_ref = refs
        x = a_ref[...] - b_ref[...]
        y = jnp.dot(x, w_ref[...], preferred_element_type=jnp.float32)
        y = y + bias_ref[...]
        res = x if res_from_x else r_ref[...]
        m = jnp.maximum(_ALPHA * y + (1.0 - _ALPHA) * res, _LN_ZERO)
        lane = lax.broadcasted_iota(jnp.int32, m.shape, 1)
        mx = m
        for sft in (1, 2, 4, 8):
            up = pltpu.roll(mx, 128 - sft, 1)
            dn = pltpu.roll(mx, sft, 1)
            mx = jnp.maximum(mx, jnp.where((lane & sft) != 0, dn, up))
        e = jnp.exp(m - mx)
        gi = lax.broadcasted_iota(jnp.int32, (128, 128), 0)
        gj = lax.broadcasted_iota(jnp.int32, (128, 128), 1)
        ones_bd = ((gi // 16) == (gj // 16)).astype(jnp.float32)
        ssum = jnp.dot(e, ones_bd, preferred_element_type=jnp.float32)
        o_ref[...] = m - mx - jnp.log(ssum)

    edge_spec = pl.BlockSpec((br, 128), lambda i: (i, 0))
    in_specs = [edge_spec, edge_spec]
    operands = [a2d, b2d]
    if not res_from_x:
        in_specs.append(edge_spec)
        operands.append(r2d)
    in_specs += [
        pl.BlockSpec((128, 128), lambda i: (0, 0)),
        pl.BlockSpec((1, 128), lambda i: (0, 0)),
    ]
    operands += [w, bias]
    return pl.pallas_call(
        body,
        grid=(rows // br,),
        in_specs=in_specs,
        out_specs=edge_spec,
        out_shape=jax.ShapeDtypeStruct((rows, 128), jnp.float32),
    )(*operands)


def _add_pair(p):
    n_rows = p.shape[1]
    a = p[0].reshape(n_rows * _S // 128, 128)
    b = p[1].reshape(n_rows * _S // 128, 128)
    rows = a.shape[0]
    br = 512
    grid = (rows + br - 1) // br

    def body(a_ref, b_ref, o_ref):
        o_ref[...] = a_ref[...] + b_ref[...]

    spec = pl.BlockSpec((br, 128), lambda i: (i, 0))
    out = pl.pallas_call(
        body,
        grid=(grid,),
        in_specs=[spec, spec],
        out_specs=spec,
        out_shape=jax.ShapeDtypeStruct((rows, 128), jnp.float32),
    )(a, b)
    return out.reshape(n_rows, _S)


def kernel(prv_varToFactor_messages, prv_factorToVar_messages, prv_factor_beliefs,
           W1, b1, W2, b2, W3, b3, W4, b4,
           factor_edge_idx, var_edge_idx):
    f_idx2 = factor_edge_idx.astype(jnp.int32).reshape(_CH, _CHUNK)
    v_idx2 = var_edge_idx.astype(jnp.int32).reshape(_CH, _CHUNK)

    wc1 = W2 @ W1
    bc1 = b1 @ W2.T + b2
    wc2 = W4 @ W3
    bc2 = b3 @ W4.T + b4
    eye8 = jnp.eye(8, dtype=jnp.float32)
    bd1 = jnp.kron(eye8, wc1.T)
    bd2 = jnp.kron(eye8, wc2.T)
    bt1 = jnp.tile(bc1, 8)[None, :]
    bt2 = jnp.tile(bc2, 8)[None, :]

    rows2d = _E * _S // 128

    fb_edges = _sc_gather(prv_factor_beliefs, f_idx2)

    ftv2d = _dense_pass(
        fb_edges.reshape(rows2d, 128),
        prv_varToFactor_messages.reshape(rows2d, 128),
        prv_factorToVar_messages.reshape(rows2d, 128),
        bd1, bt1)
    factorToVar_messages = ftv2d.reshape(_E, _S)

    vz = jnp.zeros((_NUM_VARS, _S), jnp.float32)
    vparts = _sc_scatter_add(
        ftv2d.reshape(_CH, _CHUNK, _S), v_idx2, _NUM_VARS, vz, 12)
    var_beliefs = _add_pair(vparts)

    vb_edges = _sc_gather(var_beliefs, v_idx2)

    vtf2d = _dense_pass(
        vb_edges.reshape(rows2d, 128),
        ftv2d,
        None,
        bd2, bt2)
    varToFactor_messages = vtf2d.reshape(_E, _S)

    fz = jnp.zeros((_NUM_FACTORS, _S), jnp.float32)
    fparts = _sc_scatter_add(
        vtf2d.reshape(_CH, _CHUNK, _S), f_idx2, _NUM_FACTORS, fz, 16)
    factor_beliefs = _add_pair(fparts)

    return (varToFactor_messages, factorToVar_messages, factor_beliefs, var_beliefs)


# speedup vs baseline: 3.2356x; 1.0002x over previous
import functools

import jax
import jax.numpy as jnp
from jax import lax
from jax.experimental import pallas as pl
from jax.experimental.pallas import tpu as pltpu
from jax.experimental.pallas import tpu_sc as plsc

_NUM_FACTORS = 50000
_NUM_VARS = 100000
_S = 16
_E = 1600000
_LN_ZERO = -99.0
_ALPHA = 0.5

_NW = 32
_CHUNK = 128
_CH = _E // _CHUNK
_BASE = _CH // _NW
_EXTRA = _CH % _NW


def _mesh():
    return plsc.VectorSubcoreMesh(core_axis_name="c", subcore_axis_name="s")


_SC_PARAMS = pltpu.CompilerParams(use_tc_tiling_on_sc=False)


def _worker_range(c, s):
    wid = s * 2 + c
    nch = _BASE + (wid < _EXTRA).astype(jnp.int32)
    start = wid * _BASE + jnp.minimum(wid, _EXTRA)
    return start, nch


_K = 16


def _sc_gather(table, idx2):
    nsub = 2
    sub = _K * _CHUNK // nsub

    @functools.partial(
        pl.kernel,
        mesh=_mesh(),
        out_type=jax.ShapeDtypeStruct((_E, _S), jnp.float32),
        scratch_types=[
            pltpu.VMEM((_K * _CHUNK,), jnp.int32),
            pltpu.VMEM((_K * _CHUNK, _S), jnp.float32),
            pltpu.SemaphoreType.DMA,
            pltpu.SemaphoreType.DMA,
        ],
        compiler_params=_SC_PARAMS,
    )
    def gk(table_hbm, idx_hbm, out_hbm, idx_v, rows_v, gsem, wsem):
        start, nch = _worker_range(lax.axis_index("c"), lax.axis_index("s"))
        nsup = nch // _K

        def sup(t, carry):
            off = (start + t * _K) * _CHUNK
            pltpu.sync_copy(idx_hbm.at[pl.ds(off, _K * _CHUNK)], idx_v)

            @pl.when(t > 0)
            def _wait_prev_write():
                pltpu.make_async_copy(
                    rows_v, out_hbm.at[pl.ds(off, _K * _CHUNK)], wsem).wait()

            descs = [
                pltpu.async_copy(
                    table_hbm.at[idx_v.at[pl.ds(j * sub, sub)]],
                    rows_v.at[pl.ds(j * sub, sub)], gsem)
                for j in range(nsub)
            ]
            for d in descs:
                d.wait()
            pltpu.async_copy(rows_v, out_hbm.at[pl.ds(off, _K * _CHUNK)], wsem)
            return carry

        lax.fori_loop(0, nsup, sup, jnp.int32(0))

        @pl.when(nsup > 0)
        def _drain_last_write():
            pltpu.make_async_copy(
                rows_v, out_hbm.at[pl.ds(0, _K * _CHUNK)], wsem).wait()

        def tail(i, carry):
            off = (start + nsup * _K + i) * _CHUNK
            pltpu.sync_copy(idx_hbm.at[pl.ds(off, _CHUNK)],
                            idx_v.at[pl.ds(0, _CHUNK)])
            pltpu.async_copy(
                table_hbm.at[idx_v.at[pl.ds(0, _CHUNK)]],
                rows_v.at[pl.ds(0, _CHUNK)], gsem).wait()
            pltpu.sync_copy(rows_v.at[pl.ds(0, _CHUNK)],
                            out_hbm.at[pl.ds(off, _CHUNK)])
            return carry

        lax.fori_loop(0, nch - nsup * _K, tail, jnp.int32(0))

    return gk(table, idx2)


def _sc_scatter_add(msgs3, idx2, n_rows, zeros, k):
    span = n_rows // 16

    @functools.partial(
        pl.kernel,
        mesh=_mesh(),
        out_type=jax.ShapeDtypeStruct((2, n_rows, _S), jnp.float32),
        scratch_types=[
            pltpu.VMEM((k, _CHUNK), jnp.int32),
            pltpu.VMEM((k, _CHUNK, _S), jnp.float32),
            pltpu.VMEM_SHARED((n_rows, _S), jnp.float32),
            pltpu.SemaphoreType.DMA,
        ],
        compiler_params=_SC_PARAMS,
    )
    def sk(msgs_hbm, idx_hbm, zeros_hbm, out_hbm, idx_v, rows_v, accum, asem):
        c = lax.axis_index("c")
        s = lax.axis_index("s")
        start, nch = _worker_range(c, s)
        nsup = nch // k
        pltpu.sync_copy(zeros_hbm.at[pl.ds(s * span, span)],
                        accum.at[pl.ds(s * span, span)])
        plsc.subcore_barrier()

        def sup(t, carry):
            cr = start + t * k
            pltpu.sync_copy(idx_hbm.at[pl.ds(cr, k)], idx_v)
            pltpu.sync_copy(msgs_hbm.at[pl.ds(cr, k)], rows_v)
            descs = [
                pltpu.async_copy(
                    rows_v.at[j], accum.at[idx_v.at[j]], asem, add=True)
                for j in range(k)
            ]
            for d in descs:
                d.wait()
            return carry

        lax.fori_loop(0, nsup, sup, jnp.int32(0))

        def tail(i, carry):
            cr = start + nsup * k + i
            pltpu.sync_copy(idx_hbm.at[pl.ds(cr, 1)], idx_v.at[pl.ds(0, 1)])
            pltpu.sync_copy(msgs_hbm.at[pl.ds(cr, 1)], rows_v.at[pl.ds(0, 1)])
            pltpu.async_copy(
                rows_v.at[0], accum.at[idx_v.at[0]], asem, add=True).wait()
            return carry

        lax.fori_loop(0, nch - nsup * k, tail, jnp.int32(0))
        plsc.subcore_barrier()
        pltpu.sync_copy(accum.at[pl.ds(s * span, span)],
                        out_hbm.at[c, pl.ds(s * span, span)])

    return sk(msgs3, idx2, zeros)


def _dense_pass(a2d, b2d, r2d, w, bias):
    rows = a2d.shape[0]
    br = 2000
    res_from_x = r2d is None

    def body(*refs):
        refs = list(refs)
        a_ref = refs.pop(0)
        b_ref = refs.pop(0)
        r_ref = None if res_from_x else refs.pop(0)
        w_ref, bias_ref, o_ref = refs
        x = a_ref[...] - b_ref[...]
        y = jnp.dot(x, w_ref[...], preferred_element_type=jnp.float32)
        y = y + bias_ref[...]
        res = x if res_from_x else r_ref[...]
        m = jnp.maximum(_ALPHA * y + (1.0 - _ALPHA) * res, _LN_ZERO)
        lane = lax.broadcasted_iota(jnp.int32, m.shape, 1)
        mx = m
        for sft in (1, 2, 4, 8):
            up = pltpu.roll(mx, 128 - sft, 1)
            dn = pltpu.roll(mx, sft, 1)
            mx = jnp.maximum(mx, jnp.where((lane & sft) != 0, dn, up))
        e = jnp.exp(m - mx)
        gi = lax.broadcasted_iota(jnp.int32, (128, 128), 0)
        gj = lax.broadcasted_iota(jnp.int32, (128, 128), 1)
        ones_bd = ((gi // 16) == (gj // 16)).astype(jnp.float32)
        ssum = jnp.dot(e, ones_bd, preferred_element_type=jnp.float32)
        o_ref[...] = m - mx - jnp.log(ssum)

    edge_spec = pl.BlockSpec((br, 128), lambda i: (i, 0))
    in_specs = [edge_spec, edge_spec]
    operands = [a2d, b2d]
    if not res_from_x:
        in_specs.append(edge_spec)
        operands.append(r2d)
    in_specs += [
        pl.BlockSpec((128, 128), lambda i: (0, 0)),
        pl.BlockSpec((1, 128), lambda i: (0, 0)),
    ]
    operands += [w, bias]
    return pl.pallas_call(
        body,
        grid=(rows // br,),
        in_specs=in_specs,
        out_specs=edge_spec,
        out_shape=jax.ShapeDtypeStruct((rows, 128), jnp.float32),
    )(*operands)


def _add_pair(p):
    n_rows = p.shape[1]
    a = p[0].reshape(n_rows * _S // 128, 128)
    b = p[1].reshape(n_rows * _S // 128, 128)
    rows = a.shape[0]
    br = 512
    grid = (rows + br - 1) // br

    def body(a_ref, b_ref, o_ref):
        o_ref[...] = a_ref[...] + b_ref[...]

    spec = pl.BlockSpec((br, 128), lambda i: (i, 0))
    out = pl.pallas_call(
        body,
        grid=(grid,),
        in_specs=[spec, spec],
        out_specs=spec,
        out_shape=jax.ShapeDtypeStruct((rows, 128), jnp.float32),
    )(a, b)
    return out.reshape(n_rows, _S)


def kernel(prv_varToFactor_messages, prv_factorToVar_messages, prv_factor_beliefs,
           W1, b1, W2, b2, W3, b3, W4, b4,
           factor_edge_idx, var_edge_idx):
    f_idx = factor_edge_idx.astype(jnp.int32)
    v_idx = var_edge_idx.astype(jnp.int32)
    f_idx2 = f_idx.reshape(_CH, _CHUNK)
    v_idx2 = v_idx.reshape(_CH, _CHUNK)

    wc1 = W2 @ W1
    bc1 = b1 @ W2.T + b2
    wc2 = W4 @ W3
    bc2 = b3 @ W4.T + b4
    eye8 = jnp.eye(8, dtype=jnp.float32)
    bd1 = jnp.kron(eye8, wc1.T)
    bd2 = jnp.kron(eye8, wc2.T)
    bt1 = jnp.tile(bc1, 8)[None, :]
    bt2 = jnp.tile(bc2, 8)[None, :]

    rows2d = _E * _S // 128

    fb_edges = _sc_gather(prv_factor_beliefs, f_idx)

    ftv2d = _dense_pass(
        fb_edges.reshape(rows2d, 128),
        prv_varToFactor_messages.reshape(rows2d, 128),
        prv_factorToVar_messages.reshape(rows2d, 128),
        bd1, bt1)
    factorToVar_messages = ftv2d.reshape(_E, _S)

    vz = jnp.zeros((_NUM_VARS, _S), jnp.float32)
    vparts = _sc_scatter_add(
        ftv2d.reshape(_CH, _CHUNK, _S), v_idx2, _NUM_VARS, vz, 12)
    var_beliefs = _add_pair(vparts)

    vb_edges = _sc_gather(var_beliefs, v_idx)

    vtf2d = _dense_pass(
        vb_edges.reshape(rows2d, 128),
        ftv2d,
        None,
        bd2, bt2)
    varToFactor_messages = vtf2d.reshape(_E, _S)

    fz = jnp.zeros((_NUM_FACTORS, _S), jnp.float32)
    fparts = _sc_scatter_add(
        vtf2d.reshape(_CH, _CHUNK, _S), f_idx2, _NUM_FACTORS, fz, 16)
    factor_beliefs = _add_pair(fparts)

    return (varToFactor_messages, factorToVar_messages, factor_beliefs, var_beliefs)


# speedup vs baseline: 3.2415x; 1.0018x over previous
import functools

import jax
import jax.numpy as jnp
from jax import lax
from jax.experimental import pallas as pl
from jax.experimental.pallas import tpu as pltpu
from jax.experimental.pallas import tpu_sc as plsc

_NUM_FACTORS = 50000
_NUM_VARS = 100000
_S = 16
_E = 1600000
_LN_ZERO = -99.0
_ALPHA = 0.5

_NW = 32
_CHUNK = 128
_CH = _E // _CHUNK
_BASE = _CH // _NW
_EXTRA = _CH % _NW


def _mesh():
    return plsc.VectorSubcoreMesh(core_axis_name="c", subcore_axis_name="s")


_SC_PARAMS = pltpu.CompilerParams(use_tc_tiling_on_sc=False)


def _worker_range(c, s):
    wid = s * 2 + c
    nch = _BASE + (wid < _EXTRA).astype(jnp.int32)
    start = wid * _BASE + jnp.minimum(wid, _EXTRA)
    return start, nch


_K = 24


def _sc_gather(table, idx2):
    nsub = 2
    sub = _K * _CHUNK // nsub

    @functools.partial(
        pl.kernel,
        mesh=_mesh(),
        out_type=jax.ShapeDtypeStruct((_E, _S), jnp.float32),
        scratch_types=[
            pltpu.VMEM((_K * _CHUNK,), jnp.int32),
            pltpu.VMEM((_K * _CHUNK, _S), jnp.float32),
            pltpu.SemaphoreType.DMA,
            pltpu.SemaphoreType.DMA,
        ],
        compiler_params=_SC_PARAMS,
    )
    def gk(table_hbm, idx_hbm, out_hbm, idx_v, rows_v, gsem, wsem):
        start, nch = _worker_range(lax.axis_index("c"), lax.axis_index("s"))
        nsup = nch // _K

        def sup(t, carry):
            off = (start + t * _K) * _CHUNK
            pltpu.sync_copy(idx_hbm.at[pl.ds(off, _K * _CHUNK)], idx_v)

            @pl.when(t > 0)
            def _wait_prev_write():
                pltpu.make_async_copy(
                    rows_v, out_hbm.at[pl.ds(off, _K * _CHUNK)], wsem).wait()

            descs = [
                pltpu.async_copy(
                    table_hbm.at[idx_v.at[pl.ds(j * sub, sub)]],
                    rows_v.at[pl.ds(j * sub, sub)], gsem)
                for j in range(nsub)
            ]
            for d in descs:
                d.wait()
            pltpu.async_copy(rows_v, out_hbm.at[pl.ds(off, _K * _CHUNK)], wsem)
            return carry

        lax.fori_loop(0, nsup, sup, jnp.int32(0))

        @pl.when(nsup > 0)
        def _drain_last_write():
            pltpu.make_async_copy(
                rows_v, out_hbm.at[pl.ds(0, _K * _CHUNK)], wsem).wait()

        def tail(i, carry):
            off = (start + nsup * _K + i) * _CHUNK
            pltpu.sync_copy(idx_hbm.at[pl.ds(off, _CHUNK)],
                            idx_v.at[pl.ds(0, _CHUNK)])
            pltpu.async_copy(
                table_hbm.at[idx_v.at[pl.ds(0, _CHUNK)]],
                rows_v.at[pl.ds(0, _CHUNK)], gsem).wait()
            pltpu.sync_copy(rows_v.at[pl.ds(0, _CHUNK)],
                            out_hbm.at[pl.ds(off, _CHUNK)])
            return carry

        lax.fori_loop(0, nch - nsup * _K, tail, jnp.int32(0))

    return gk(table, idx2)


def _sc_scatter_add(msgs3, idx2, n_rows, zeros, k):
    span = n_rows // 16

    @functools.partial(
        pl.kernel,
        mesh=_mesh(),
        out_type=jax.ShapeDtypeStruct((2, n_rows, _S), jnp.float32),
        scratch_types=[
            pltpu.VMEM((k, _CHUNK), jnp.int32),
            pltpu.VMEM((k, _CHUNK, _S), jnp.float32),
            pltpu.VMEM_SHARED((n_rows, _S), jnp.float32),
            pltpu.SemaphoreType.DMA,
        ],
        compiler_params=_SC_PARAMS,
    )
    def sk(msgs_hbm, idx_hbm, zeros_hbm, out_hbm, idx_v, rows_v, accum, asem):
        c = lax.axis_index("c")
        s = lax.axis_index("s")
        start, nch = _worker_range(c, s)
        nsup = nch // k
        pltpu.sync_copy(zeros_hbm.at[pl.ds(s * span, span)],
                        accum.at[pl.ds(s * span, span)])
        plsc.subcore_barrier()

        def sup(t, carry):
            cr = start + t * k
            pltpu.sync_copy(idx_hbm.at[pl.ds(cr, k)], idx_v)
            pltpu.sync_copy(msgs_hbm.at[pl.ds(cr, k)], rows_v)
            descs = [
                pltpu.async_copy(
                    rows_v.at[j], accum.at[idx_v.at[j]], asem, add=True)
                for j in range(k)
            ]
            for d in descs:
                d.wait()
            return carry

        lax.fori_loop(0, nsup, sup, jnp.int32(0))

        def tail(i, carry):
            cr = start + nsup * k + i
            pltpu.sync_copy(idx_hbm.at[pl.ds(cr, 1)], idx_v.at[pl.ds(0, 1)])
            pltpu.sync_copy(msgs_hbm.at[pl.ds(cr, 1)], rows_v.at[pl.ds(0, 1)])
            pltpu.async_copy(
                rows_v.at[0], accum.at[idx_v.at[0]], asem, add=True).wait()
            return carry

        lax.fori_loop(0, nch - nsup * k, tail, jnp.int32(0))
        plsc.subcore_barrier()
        pltpu.sync_copy(accum.at[pl.ds(s * span, span)],
                        out_hbm.at[c, pl.ds(s * span, span)])

    return sk(msgs3, idx2, zeros)


def _dense_pass(a2d, b2d, r2d, w, bias):
    rows = a2d.shape[0]
    br = 2000
    res_from_x = r2d is None

    def body(*refs):
        refs = list(refs)
        a_ref = refs.pop(0)
        b_ref = refs.pop(0)
        r_ref = None if res_from_x else refs.pop(0)
        w_ref, bias_ref, o_ref = refs
        x = a_ref[...] - b_ref[...]
        y = jnp.dot(x, w_ref[...], preferred_element_type=jnp.float32)
        y = y + bias_ref[...]
        res = x if res_from_x else r_ref[...]
        m = jnp.maximum(_ALPHA * y + (1.0 - _ALPHA) * res, _LN_ZERO)
        lane = lax.broadcasted_iota(jnp.int32, m.shape, 1)
        mx = m
        for sft in (1, 2, 4, 8):
            up = pltpu.roll(mx, 128 - sft, 1)
            dn = pltpu.roll(mx, sft, 1)
            mx = jnp.maximum(mx, jnp.where((lane & sft) != 0, dn, up))
        e = jnp.exp(m - mx)
        gi = lax.broadcasted_iota(jnp.int32, (128, 128), 0)
        gj = lax.broadcasted_iota(jnp.int32, (128, 128), 1)
        ones_bd = ((gi // 16) == (gj // 16)).astype(jnp.float32)
        ssum = jnp.dot(e, ones_bd, preferred_element_type=jnp.float32)
        o_ref[...] = m - mx - jnp.log(ssum)

    edge_spec = pl.BlockSpec((br, 128), lambda i: (i, 0))
    in_specs = [edge_spec, edge_spec]
    operands = [a2d, b2d]
    if not res_from_x:
        in_specs.append(edge_spec)
        operands.append(r2d)
    in_specs += [
        pl.BlockSpec((128, 128), lambda i: (0, 0)),
        pl.BlockSpec((1, 128), lambda i: (0, 0)),
    ]
    operands += [w, bias]
    return pl.pallas_call(
        body,
        grid=(rows // br,),
        in_specs=in_specs,
        out_specs=edge_spec,
        out_shape=jax.ShapeDtypeStruct((rows, 128), jnp.float32),
    )(*operands)


def _add_pair(p):
    n_rows = p.shape[1]
    a = p[0].reshape(n_rows * _S // 128, 128)
    b = p[1].reshape(n_rows * _S // 128, 128)
    rows = a.shape[0]
    br = 512
    grid = (rows + br - 1) // br

    def body(a_ref, b_ref, o_ref):
        o_ref[...] = a_ref[...] + b_ref[...]

    spec = pl.BlockSpec((br, 128), lambda i: (i, 0))
    out = pl.pallas_call(
        body,
        grid=(grid,),
        in_specs=[spec, spec],
        out_specs=spec,
        out_shape=jax.ShapeDtypeStruct((rows, 128), jnp.float32),
    )(a, b)
    return out.reshape(n_rows, _S)


def kernel(prv_varToFactor_messages, prv_factorToVar_messages, prv_factor_beliefs,
           W1, b1, W2, b2, W3, b3, W4, b4,
           factor_edge_idx, var_edge_idx):
    f_idx = factor_edge_idx.astype(jnp.int32)
    v_idx = var_edge_idx.astype(jnp.int32)
    f_idx2 = f_idx.reshape(_CH, _CHUNK)
    v_idx2 = v_idx.reshape(_CH, _CHUNK)

    wc1 = W2 @ W1
    bc1 = b1 @ W2.T + b2
    wc2 = W4 @ W3
    bc2 = b3 @ W4.T + b4
    eye8 = jnp.eye(8, dtype=jnp.float32)
    bd1 = jnp.kron(eye8, wc1.T)
    bd2 = jnp.kron(eye8, wc2.T)
    bt1 = jnp.tile(bc1, 8)[None, :]
    bt2 = jnp.tile(bc2, 8)[None, :]

    rows2d = _E * _S // 128

    fb_edges = _sc_gather(prv_factor_beliefs, f_idx)

    ftv2d = _dense_pass(
        fb_edges.reshape(rows2d, 128),
        prv_varToFactor_messages.reshape(rows2d, 128),
        prv_factorToVar_messages.reshape(rows2d, 128),
        bd1, bt1)
    factorToVar_messages = ftv2d.reshape(_E, _S)

    vz = jnp.zeros((_NUM_VARS, _S), jnp.float32)
    vparts = _sc_scatter_add(
        ftv2d.reshape(_CH, _CHUNK, _S), v_idx2, _NUM_VARS, vz, 12)
    var_beliefs = _add_pair(vparts)

    vb_edges = _sc_gather(var_beliefs, v_idx)

    vtf2d = _dense_pass(
        vb_edges.reshape(rows2d, 128),
        ftv2d,
        None,
        bd2, bt2)
    varToFactor_messages = vtf2d.reshape(_E, _S)

    fz = jnp.zeros((_NUM_FACTORS, _S), jnp.float32)
    fparts = _sc_scatter_add(
        vtf2d.reshape(_CH, _CHUNK, _S), f_idx2, _NUM_FACTORS, fz, 24)
    factor_beliefs = _add_pair(fparts)

    return (varToFactor_messages, factorToVar_messages, factor_beliefs, var_beliefs)
